# baseline (speedup 1.0000x reference)
import jax
import jax.numpy as jnp
from jax import lax
from jax.experimental import pallas as pl
from jax.experimental.pallas import tpu as pltpu
from jax.experimental.pallas import tpu_sc as plsc

B, S, H = 1, 2048, 768
E = 8
F_INT = 1024
S_INT = 512

T = B * S
TILE_M = 128
P = T * 2 + E * TILE_M
NPAD = P - 2 * T
NT2 = P // TILE_M
TE_LEN = 48

NW1 = 16
TW1 = T // NW1
NW2 = 32
RW2 = P // NW2
GCHUNK = 80
TW3 = T // NW2
CCHUNK = 32

ROUTER_TILE = 512
N_RT = T // ROUTER_TILE


def _k1_router_shared(x_ref, gate_w_ref, sg_ref, su_ref, sd_ref,
                      rout_ref, shared_ref, cnt_ref, cacc_ref):
    i = pl.program_id(0)
    x = x_ref[...]
    lt = lax.dot_general(gate_w_ref[...], x, (((0,), (1,)), ((), ())),
                         preferred_element_type=jnp.float32)
    m1 = jnp.max(lt, axis=0, keepdims=True)
    i1 = jnp.argmax(lt, axis=0, keepdims=True)
    eids = lax.broadcasted_iota(jnp.int32, lt.shape, 0)
    masked = jnp.where(eids == i1, -jnp.inf, lt)
    m2 = jnp.max(masked, axis=0, keepdims=True)
    i2 = jnp.argmax(masked, axis=0, keepdims=True)
    p1 = 1.0 / (1.0 + jnp.exp(m2 - m1))
    rout_ref[...] = jnp.concatenate(
        [i1.astype(jnp.float32), i2.astype(jnp.float32), p1, 1.0 - p1], axis=0)

    oh = (jnp.where(eids == i1, 1.0, 0.0)
          + jnp.where(eids == i2, 1.0, 0.0))
    ct = jnp.sum(oh, axis=1, keepdims=True)

    @pl.when(i == 0)
    def _():
        cacc_ref[...] = jnp.zeros_like(cacc_ref)

    cacc_ref[:, 0:1] += ct
    cnt_ref[...] = cacc_ref[:, 0:1]

    sg = jnp.dot(x, sg_ref[...], preferred_element_type=jnp.float32)
    su = jnp.dot(x, su_ref[...], preferred_element_type=jnp.float32)
    hmid = (sg * lax.logistic(sg)) * su
    shared_ref[...] = jnp.dot(hmid, sd_ref[...],
                              preferred_element_type=jnp.float32)


def _scan_lanes(v, n):
    sh = 1
    while sh < n:
        z = jnp.zeros(v.shape[:-1] + (sh,), v.dtype)
        v = v + jnp.concatenate([z, v[..., :-sh]], axis=-1)
        sh *= 2
    return v


def _scan_sublanes(v, n):
    sh = 1
    while sh < n:
        z = jnp.zeros((sh,) + v.shape[1:], v.dtype)
        v = v + jnp.concatenate([z, v[:-sh]], axis=0)
        sh *= 2
    return v


def _k1b_dispatch(rout_ref, cnt_ref, pos_ref, pad_ref, te_ref, carry_ref):
    i = pl.program_id(0)
    c = cnt_ref[...]
    rounded = jnp.floor((c + (TILE_M - 1.0)) * (1.0 / TILE_M)) * TILE_M
    csum = _scan_sublanes(rounded, E)
    excl = csum - rounded

    i1 = rout_ref[0:1, :]
    i2 = rout_ref[1:2, :]
    eids = lax.broadcasted_iota(jnp.int32, (E, ROUTER_TILE), 0).astype(jnp.float32)
    oh0 = jnp.where(eids == i1, 1.0, 0.0)
    oh1 = jnp.where(eids == i2, 1.0, 0.0)

    @pl.when(i == 0)
    def _():
        carry_ref[...] = jnp.zeros_like(carry_ref)
        carry_ref[:, 0:1] += excl

    carry = carry_ref[:, 0:1]
    cs0 = _scan_lanes(oh0, ROUTER_TILE)
    cs1 = _scan_lanes(oh1, ROUTER_TILE)
    m = carry + (cs0 - oh0) + (cs1 - oh1)
    d0 = jnp.sum(oh0 * m, axis=0, keepdims=True)
    d1 = jnp.sum(oh1 * (m + oh0), axis=0, keepdims=True)
    pos_ref[...] = jnp.concatenate([d0, d1], axis=0).astype(jnp.int32)
    carry_ref[:, 0:1] += jnp.sum(oh0 + oh1, axis=1, keepdims=True)

    @pl.when(i == 0)
    def _():
        padsz = rounded - c
        pcum = _scan_sublanes(padsz, E)
        exc_s = [excl[e:e + 1, 0:1] for e in range(E)] + [csum[E - 1:E, 0:1]]
        tot_s = [c[e:e + 1, 0:1] for e in range(E)] + [jnp.zeros((1, 1))]
        pci_s = [pcum[e:e + 1, 0:1] for e in range(E)] + [
            jnp.full((1, 1), float(NPAD))]
        pce_s = [pcum[e:e + 1, 0:1] - padsz[e:e + 1, 0:1]
                 for e in range(E)] + [pcum[E - 1:E, 0:1]]

        wrow = lax.broadcasted_iota(jnp.int32, (NW1, TILE_M), 0).astype(jnp.float32)
        lcol = lax.broadcasted_iota(jnp.int32, (NW1, TILE_M), 1).astype(jnp.float32)
        halfw = float(NPAD // NW1)
        pv = wrow * halfw + lcol
        esel = jnp.zeros((NW1, TILE_M))
        for e in range(E + 1):
            esel = esel + jnp.where(pv >= pci_s[e], 1.0, 0.0)
        padpos = jnp.zeros((NW1, TILE_M))
        for e in range(E + 1):
            padpos = padpos + jnp.where(
                esel == float(e), exc_s[e] + tot_s[e] - pce_s[e], 0.0)
        padpos = padpos + pv
        trash = float(P) + wrow * halfw + (lcol - halfw)
        pad_ref[...] = jnp.where(lcol < halfw, padpos, trash).astype(jnp.int32)

        tcol = lax.broadcasted_iota(jnp.int32, (1, TE_LEN), 1).astype(jnp.float32) * TILE_M
        acc = jnp.zeros((1, TE_LEN))
        for e in range(E):
            acc = acc + jnp.where(tcol >= csum[e:e + 1, 0:1], 1.0, 0.0)
        te_ref[...] = jnp.minimum(acc, E - 1.0).astype(jnp.int32)


def _sca_scatter(rout_hbm, pos_hbm, pad_hbm, tok_hbm, prob_hbm,
                 destb_v, tokb_v, probb_v, sem):
    w = lax.axis_index("s")
    iota16 = lax.iota(jnp.int32, 16)
    z16i = jnp.zeros((16,), jnp.int32)
    z16f = jnp.zeros((16,), jnp.float32)

    for s in (0, 1):
        pltpu.sync_copy(pos_hbm.at[s, pl.ds(w * TW1, TW1)], destb_v.at[s])
        pltpu.sync_copy(rout_hbm.at[2 + s, pl.ds(w * TW1, TW1)],
                        probb_v.at[s])
    pltpu.sync_copy(pad_hbm.at[w], destb_v.at[2])
    for k in range(TW1 // 16):
        tv = jnp.full((16,), w * TW1 + k * 16, jnp.int32) + iota16
        tokb_v[0, pl.ds(k * 16, 16)] = tv
        tokb_v[1, pl.ds(k * 16, 16)] = tv
        tokb_v[2, pl.ds(k * 16, 16)] = z16i
        probb_v[2, pl.ds(k * 16, 16)] = z16f

    for s in (0, 1, 2):
        pltpu.async_copy(tokb_v.at[s], tok_hbm.at[destb_v.at[s]], sem).wait()
        pltpu.async_copy(probb_v.at[s], prob_hbm.at[destb_v.at[s]], sem).wait()


def _scb_gather(x_hbm, tok_hbm, xs_hbm, idx_v, rows_v, sem):
    w = lax.axis_index("s") * 2 + lax.axis_index("c")
    pltpu.sync_copy(tok_hbm.at[pl.ds(w * RW2, RW2)], idx_v)
    for c in range(RW2 // GCHUNK):
        pltpu.async_copy(x_hbm.at[idx_v.at[pl.ds(c * GCHUNK, GCHUNK)]],
                         rows_v, sem).wait()
        pltpu.sync_copy(rows_v, xs_hbm.at[pl.ds(w * RW2 + c * GCHUNK, GCHUNK)])


def _scc_combine(outs_hbm, pos_hbm, shared_hbm, y_hbm,
                 pos0_v, pos1_v, r0_v, r1_v, sh_v, sem):
    w = lax.axis_index("s") * 2 + lax.axis_index("c")
    pltpu.sync_copy(pos_hbm.at[0, pl.ds(w * TW3, TW3)], pos0_v)
    pltpu.sync_copy(pos_hbm.at[1, pl.ds(w * TW3, TW3)], pos1_v)
    for c in range(TW3 // CCHUNK):
        pltpu.async_copy(outs_hbm.at[pos0_v.at[pl.ds(c * CCHUNK, CCHUNK)]],
                         r0_v, sem).wait()
        pltpu.async_copy(outs_hbm.at[pos1_v.at[pl.ds(c * CCHUNK, CCHUNK)]],
                         r1_v, sem).wait()
        pltpu.sync_copy(shared_hbm.at[pl.ds(w * TW3 + c * CCHUNK, CCHUNK)],
                        sh_v)

        def row_body(r, _):
            for k in range(H // 16):
                sl = pl.ds(k * 16, 16)
                r0_v[r, sl] = r0_v[r, sl] + r1_v[r, sl] + sh_v[r, sl]
            return 0

        lax.fori_loop(0, CCHUNK, row_body, 0)
        pltpu.sync_copy(r0_v, y_hbm.at[pl.ds(w * TW3 + c * CCHUNK, CCHUNK)])


def _k2_grouped(te_ref, xs_ref, wg_ref, wu_ref, wd_ref, prob_ref, out_ref):
    xs = xs_ref[...]
    g = jnp.dot(xs, wg_ref[0], preferred_element_type=jnp.float32)
    u = jnp.dot(xs, wu_ref[0], preferred_element_type=jnp.float32)
    hmid = (g * lax.logistic(g)) * u
    eo = jnp.dot(hmid, wd_ref[0], preferred_element_type=jnp.float32)
    out_ref[...] = eo * prob_ref[...]


@jax.jit
def kernel(hidden_states, gate_w, w_gate, w_up, w_down, s_gate, s_up, s_down):
    b, s, h = hidden_states.shape
    x = hidden_states.reshape(-1, h)

    rout, shared, cnts = pl.pallas_call(
        _k1_router_shared,
        grid=(N_RT,),
        in_specs=[
            pl.BlockSpec((ROUTER_TILE, H), lambda t: (t, 0)),
            pl.BlockSpec((H, E), lambda t: (0, 0)),
            pl.BlockSpec((H, S_INT), lambda t: (0, 0)),
            pl.BlockSpec((H, S_INT), lambda t: (0, 0)),
            pl.BlockSpec((S_INT, H), lambda t: (0, 0)),
        ],
        out_specs=[
            pl.BlockSpec((4, ROUTER_TILE), lambda t: (0, t)),
            pl.BlockSpec((ROUTER_TILE, H), lambda t: (t, 0)),
            pl.BlockSpec((E, 1), lambda t: (0, 0)),
        ],
        out_shape=[
            jax.ShapeDtypeStruct((4, T), jnp.float32),
            jax.ShapeDtypeStruct((T, H), jnp.float32),
            jax.ShapeDtypeStruct((E, 1), jnp.float32),
        ],
        scratch_shapes=[pltpu.VMEM((E, 128), jnp.float32)],
    )(x, gate_w, s_gate, s_up, s_down)

    pos, pad, te = pl.pallas_call(
        _k1b_dispatch,
        grid=(N_RT,),
        in_specs=[
            pl.BlockSpec((4, ROUTER_TILE), lambda t: (0, t)),
            pl.BlockSpec((E, 1), lambda t: (0, 0)),
        ],
        out_specs=[
            pl.BlockSpec((2, ROUTER_TILE), lambda t: (0, t)),
            pl.BlockSpec((NW1, TILE_M), lambda t: (0, 0)),
            pl.BlockSpec((1, TE_LEN), lambda t: (0, 0)),
        ],
        out_shape=[
            jax.ShapeDtypeStruct((2, T), jnp.int32),
            jax.ShapeDtypeStruct((NW1, TILE_M), jnp.int32),
            jax.ShapeDtypeStruct((1, TE_LEN), jnp.int32),
        ],
        scratch_shapes=[pltpu.VMEM((E, 128), jnp.float32)],
    )(rout, cnts)

    mesh1 = plsc.VectorSubcoreMesh(core_axis_name="c", subcore_axis_name="s",
                                   num_cores=1)
    scatter = pl.kernel(
        _sca_scatter,
        out_type=(
            jax.ShapeDtypeStruct((P + NPAD,), jnp.int32),
            jax.ShapeDtypeStruct((P + NPAD,), jnp.float32),
        ),
        mesh=mesh1,
        compiler_params=pltpu.CompilerParams(needs_layout_passes=False),
        scratch_types=[
            pltpu.VMEM((3, TW1), jnp.int32),
            pltpu.VMEM((3, TW1), jnp.int32),
            pltpu.VMEM((3, TW1), jnp.float32),
            pltpu.SemaphoreType.DMA,
        ],
    )
    tok_sorted, prob_sorted = scatter(rout, pos, pad)
    tok_sorted = tok_sorted[:P]
    prob_sorted = prob_sorted[:P]

    mesh2 = plsc.VectorSubcoreMesh(core_axis_name="c", subcore_axis_name="s")
    gather = pl.kernel(
        _scb_gather,
        out_type=jax.ShapeDtypeStruct((P, H), jnp.float32),
        mesh=mesh2,
        compiler_params=pltpu.CompilerParams(needs_layout_passes=False),
        scratch_types=[
            pltpu.VMEM((RW2,), jnp.int32),
            pltpu.VMEM((GCHUNK, H), jnp.float32),
            pltpu.SemaphoreType.DMA,
        ],
    )
    xs = gather(x, tok_sorted)

    outs = pl.pallas_call(
        _k2_grouped,
        grid_spec=pltpu.PrefetchScalarGridSpec(
            num_scalar_prefetch=1,
            grid=(NT2,),
            in_specs=[
                pl.BlockSpec((TILE_M, H), lambda i, te: (i, 0)),
                pl.BlockSpec((1, H, F_INT), lambda i, te: (te[i], 0, 0)),
                pl.BlockSpec((1, H, F_INT), lambda i, te: (te[i], 0, 0)),
                pl.BlockSpec((1, F_INT, H), lambda i, te: (te[i], 0, 0)),
                pl.BlockSpec((TILE_M, 1), lambda i, te: (i, 0)),
            ],
            out_specs=pl.BlockSpec((TILE_M, H), lambda i, te: (i, 0)),
        ),
        out_shape=jax.ShapeDtypeStruct((P, H), jnp.float32),
    )(te.reshape(TE_LEN), xs, w_gate, w_up, w_down, prob_sorted.reshape(P, 1))

    combine = pl.kernel(
        _scc_combine,
        out_type=jax.ShapeDtypeStruct((T, H), jnp.float32),
        mesh=mesh2,
        compiler_params=pltpu.CompilerParams(needs_layout_passes=False),
        scratch_types=[
            pltpu.VMEM((TW3,), jnp.int32),
            pltpu.VMEM((TW3,), jnp.int32),
            pltpu.VMEM((CCHUNK, H), jnp.float32),
            pltpu.VMEM((CCHUNK, H), jnp.float32),
            pltpu.VMEM((CCHUNK, H), jnp.float32),
            pltpu.SemaphoreType.DMA,
        ],
    )
    y = combine(outs, pos, shared)
    return y.reshape(b, s, h)


# speedup vs baseline: 1.5802x; 1.5802x over previous
import jax
import jax.numpy as jnp
from jax import lax
from jax.experimental import pallas as pl
from jax.experimental.pallas import tpu as pltpu
from jax.experimental.pallas import tpu_sc as plsc

B, S, H = 1, 2048, 768
E = 8
F_INT = 1024
S_INT = 512

T = B * S
TILE_M = 128
P = T * 2 + E * TILE_M
NPAD = P - 2 * T
NT2 = P // TILE_M
TE_LEN = 48

NW1 = 16
TW1 = T // NW1
NW2 = 32
RW2 = P // NW2
GCHUNK = 80
TW3 = T // NW2
CCHUNK = 32

ROUTER_TILE = 512
N_RT = T // ROUTER_TILE


def _k1_router_shared(x_ref, gate_w_ref, sg_ref, su_ref, sd_ref,
                      rout_ref, shared_ref, cnt_ref, cacc_ref):
    i = pl.program_id(0)
    x = x_ref[...]
    lt = lax.dot_general(gate_w_ref[...], x, (((0,), (1,)), ((), ())),
                         preferred_element_type=jnp.float32)
    m1 = jnp.max(lt, axis=0, keepdims=True)
    i1 = jnp.argmax(lt, axis=0, keepdims=True)
    eids = lax.broadcasted_iota(jnp.int32, lt.shape, 0)
    masked = jnp.where(eids == i1, -jnp.inf, lt)
    m2 = jnp.max(masked, axis=0, keepdims=True)
    i2 = jnp.argmax(masked, axis=0, keepdims=True)
    p1 = 1.0 / (1.0 + jnp.exp(m2 - m1))
    rout_ref[...] = jnp.concatenate(
        [i1.astype(jnp.float32), i2.astype(jnp.float32), p1, 1.0 - p1], axis=0)

    oh = (jnp.where(eids == i1, 1.0, 0.0)
          + jnp.where(eids == i2, 1.0, 0.0))
    ct = jnp.sum(oh, axis=1, keepdims=True)

    @pl.when(i == 0)
    def _():
        cacc_ref[...] = jnp.zeros_like(cacc_ref)

    cacc_ref[:, 0:1] += ct
    cnt_ref[...] = cacc_ref[:, 0:1]

    sg = jnp.dot(x, sg_ref[...], preferred_element_type=jnp.float32)
    su = jnp.dot(x, su_ref[...], preferred_element_type=jnp.float32)
    hmid = (sg * lax.logistic(sg)) * su
    shared_ref[...] = jnp.dot(hmid, sd_ref[...],
                              preferred_element_type=jnp.float32)


def _scan_lanes(v, n):
    sh = 1
    while sh < n:
        z = jnp.zeros(v.shape[:-1] + (sh,), v.dtype)
        v = v + jnp.concatenate([z, v[..., :-sh]], axis=-1)
        sh *= 2
    return v


def _scan_sublanes(v, n):
    sh = 1
    while sh < n:
        z = jnp.zeros((sh,) + v.shape[1:], v.dtype)
        v = v + jnp.concatenate([z, v[:-sh]], axis=0)
        sh *= 2
    return v


def _k1b_dispatch(rout_ref, cnt_ref, pos_ref, te_ref, carry_ref):
    i = pl.program_id(0)
    c = cnt_ref[...]
    rounded = jnp.floor((c + (TILE_M - 1.0)) * (1.0 / TILE_M)) * TILE_M
    csum = _scan_sublanes(rounded, E)
    excl = csum - rounded

    i1 = rout_ref[0:1, :]
    i2 = rout_ref[1:2, :]
    eids = lax.broadcasted_iota(jnp.int32, (E, ROUTER_TILE), 0).astype(jnp.float32)
    oh0 = jnp.where(eids == i1, 1.0, 0.0)
    oh1 = jnp.where(eids == i2, 1.0, 0.0)

    @pl.when(i == 0)
    def _():
        carry_ref[...] = jnp.zeros_like(carry_ref)
        carry_ref[:, 0:1] += excl

    carry = carry_ref[:, 0:1]
    cs0 = _scan_lanes(oh0, ROUTER_TILE)
    cs1 = _scan_lanes(oh1, ROUTER_TILE)
    m = carry + (cs0 - oh0) + (cs1 - oh1)
    d0 = jnp.sum(oh0 * m, axis=0, keepdims=True)
    d1 = jnp.sum(oh1 * (m + oh0), axis=0, keepdims=True)
    pos_ref[...] = jnp.concatenate([d0, d1], axis=0).astype(jnp.int32)
    carry_ref[:, 0:1] += jnp.sum(oh0 + oh1, axis=1, keepdims=True)

    @pl.when(i == 0)
    def _():
        tcol = lax.broadcasted_iota(jnp.int32, (1, TE_LEN), 1).astype(jnp.float32) * TILE_M
        acc = jnp.zeros((1, TE_LEN))
        for e in range(E):
            acc = acc + jnp.where(tcol >= csum[e:e + 1, 0:1], 1.0, 0.0)
        te_ref[...] = jnp.minimum(acc, E - 1.0).astype(jnp.int32)


def _scb_rowscatter(x_hbm, rout_hbm, pos_hbm, xs_hbm, prob_hbm,
                    pos0_v, pos1_v, xrows_v, pb0_v, pb1_v, semA, semB):
    w = lax.axis_index("s") * 2 + lax.axis_index("c")
    pltpu.sync_copy(pos_hbm.at[0, pl.ds(w * TW3, TW3)], pos0_v)
    pltpu.sync_copy(pos_hbm.at[1, pl.ds(w * TW3, TW3)], pos1_v)
    pltpu.sync_copy(x_hbm.at[pl.ds(w * TW3, TW3)], xrows_v)
    a1 = pltpu.async_copy(xrows_v, xs_hbm.at[pos0_v], semA)
    a2 = pltpu.async_copy(xrows_v, xs_hbm.at[pos1_v], semB)
    a1.wait()
    a2.wait()
    pltpu.sync_copy(rout_hbm.at[2, pl.ds(w * TW3, TW3)], pb0_v)
    pltpu.sync_copy(rout_hbm.at[3, pl.ds(w * TW3, TW3)], pb1_v)
    a3 = pltpu.async_copy(pb0_v, prob_hbm.at[pos0_v], semA)
    a4 = pltpu.async_copy(pb1_v, prob_hbm.at[pos1_v], semB)
    a3.wait()
    a4.wait()


def _scc_combine(outs_hbm, pos_hbm, shared_hbm, y_hbm,
                 pos0_v, pos1_v, r0_v, r1_v, sh_v, semA, semB, semC):
    w = lax.axis_index("s") * 2 + lax.axis_index("c")
    pltpu.sync_copy(pos_hbm.at[0, pl.ds(w * TW3, TW3)], pos0_v)
    pltpu.sync_copy(pos_hbm.at[1, pl.ds(w * TW3, TW3)], pos1_v)
    for c in range(TW3 // CCHUNK):
        a1 = pltpu.async_copy(outs_hbm.at[pos0_v.at[pl.ds(c * CCHUNK, CCHUNK)]],
                              r0_v, semA)
        a2 = pltpu.async_copy(outs_hbm.at[pos1_v.at[pl.ds(c * CCHUNK, CCHUNK)]],
                              r1_v, semB)
        a3 = pltpu.async_copy(
            shared_hbm.at[pl.ds(w * TW3 + c * CCHUNK, CCHUNK)], sh_v, semC)
        a1.wait()
        a2.wait()
        a3.wait()

        def row_body(r, _):
            for k in range(H // 16):
                sl = pl.ds(k * 16, 16)
                r0_v[r, sl] = r0_v[r, sl] + r1_v[r, sl] + sh_v[r, sl]
            return 0

        lax.fori_loop(0, CCHUNK, row_body, 0)
        pltpu.sync_copy(r0_v, y_hbm.at[pl.ds(w * TW3 + c * CCHUNK, CCHUNK)])


def _k2_grouped(te_ref, xs_ref, wg_ref, wu_ref, wd_ref, prob_ref, out_ref):
    xs = xs_ref[...]
    g = jnp.dot(xs, wg_ref[0], preferred_element_type=jnp.float32)
    u = jnp.dot(xs, wu_ref[0], preferred_element_type=jnp.float32)
    hmid = (g * lax.logistic(g)) * u
    eo = jnp.dot(hmid, wd_ref[0], preferred_element_type=jnp.float32)
    out_ref[...] = eo * prob_ref[...]


@jax.jit
def kernel(hidden_states, gate_w, w_gate, w_up, w_down, s_gate, s_up, s_down):
    b, s, h = hidden_states.shape
    x = hidden_states.reshape(-1, h)

    rout, shared, cnts = pl.pallas_call(
        _k1_router_shared,
        grid=(N_RT,),
        in_specs=[
            pl.BlockSpec((ROUTER_TILE, H), lambda t: (t, 0)),
            pl.BlockSpec((H, E), lambda t: (0, 0)),
            pl.BlockSpec((H, S_INT), lambda t: (0, 0)),
            pl.BlockSpec((H, S_INT), lambda t: (0, 0)),
            pl.BlockSpec((S_INT, H), lambda t: (0, 0)),
        ],
        out_specs=[
            pl.BlockSpec((4, ROUTER_TILE), lambda t: (0, t)),
            pl.BlockSpec((ROUTER_TILE, H), lambda t: (t, 0)),
            pl.BlockSpec((E, 1), lambda t: (0, 0)),
        ],
        out_shape=[
            jax.ShapeDtypeStruct((4, T), jnp.float32),
            jax.ShapeDtypeStruct((T, H), jnp.float32),
            jax.ShapeDtypeStruct((E, 1), jnp.float32),
        ],
        scratch_shapes=[pltpu.VMEM((E, 128), jnp.float32)],
    )(x, gate_w, s_gate, s_up, s_down)

    pos, te = pl.pallas_call(
        _k1b_dispatch,
        grid=(N_RT,),
        in_specs=[
            pl.BlockSpec((4, ROUTER_TILE), lambda t: (0, t)),
            pl.BlockSpec((E, 1), lambda t: (0, 0)),
        ],
        out_specs=[
            pl.BlockSpec((2, ROUTER_TILE), lambda t: (0, t)),
            pl.BlockSpec((1, TE_LEN), lambda t: (0, 0)),
        ],
        out_shape=[
            jax.ShapeDtypeStruct((2, T), jnp.int32),
            jax.ShapeDtypeStruct((1, TE_LEN), jnp.int32),
        ],
        scratch_shapes=[pltpu.VMEM((E, 128), jnp.float32)],
    )(rout, cnts)

    mesh2 = plsc.VectorSubcoreMesh(core_axis_name="c", subcore_axis_name="s")
    rowscatter = pl.kernel(
        _scb_rowscatter,
        out_type=(
            jax.ShapeDtypeStruct((P, H), jnp.float32),
            jax.ShapeDtypeStruct((P,), jnp.float32),
        ),
        mesh=mesh2,
        compiler_params=pltpu.CompilerParams(needs_layout_passes=False),
        scratch_types=[
            pltpu.VMEM((TW3,), jnp.int32),
            pltpu.VMEM((TW3,), jnp.int32),
            pltpu.VMEM((TW3, H), jnp.float32),
            pltpu.VMEM((TW3,), jnp.float32),
            pltpu.VMEM((TW3,), jnp.float32),
            pltpu.SemaphoreType.DMA,
            pltpu.SemaphoreType.DMA,
        ],
    )
    xs, prob_sorted = rowscatter(x, rout, pos)

    outs = pl.pallas_call(
        _k2_grouped,
        grid_spec=pltpu.PrefetchScalarGridSpec(
            num_scalar_prefetch=1,
            grid=(NT2,),
            in_specs=[
                pl.BlockSpec((TILE_M, H), lambda i, te: (i, 0)),
                pl.BlockSpec((1, H, F_INT), lambda i, te: (te[i], 0, 0)),
                pl.BlockSpec((1, H, F_INT), lambda i, te: (te[i], 0, 0)),
                pl.BlockSpec((1, F_INT, H), lambda i, te: (te[i], 0, 0)),
                pl.BlockSpec((TILE_M, 1), lambda i, te: (i, 0)),
            ],
            out_specs=pl.BlockSpec((TILE_M, H), lambda i, te: (i, 0)),
        ),
        out_shape=jax.ShapeDtypeStruct((P, H), jnp.float32),
    )(te.reshape(TE_LEN), xs, w_gate, w_up, w_down, prob_sorted.reshape(P, 1))

    combine = pl.kernel(
        _scc_combine,
        out_type=jax.ShapeDtypeStruct((T, H), jnp.float32),
        mesh=mesh2,
        compiler_params=pltpu.CompilerParams(needs_layout_passes=False),
        scratch_types=[
            pltpu.VMEM((TW3,), jnp.int32),
            pltpu.VMEM((TW3,), jnp.int32),
            pltpu.VMEM((CCHUNK, H), jnp.float32),
            pltpu.VMEM((CCHUNK, H), jnp.float32),
            pltpu.VMEM((CCHUNK, H), jnp.float32),
            pltpu.SemaphoreType.DMA,
            pltpu.SemaphoreType.DMA,
            pltpu.SemaphoreType.DMA,
        ],
    )
    y = combine(outs, pos, shared)
    return y.reshape(b, s, h)


# speedup vs baseline: 1.6098x; 1.0187x over previous
import jax
import jax.numpy as jnp
from jax import lax
from jax.experimental import pallas as pl
from jax.experimental.pallas import tpu as pltpu
from jax.experimental.pallas import tpu_sc as plsc

B, S, H = 1, 2048, 768
E = 8
F_INT = 1024
S_INT = 512

T = B * S
TILE_M = 128
P = T * 2 + E * TILE_M
NPAD = P - 2 * T
NT2 = P // TILE_M
TE_LEN = 48

NW1 = 16
TW1 = T // NW1
NW2 = 32
RW2 = P // NW2
GCHUNK = 80
TW3 = T // NW2
CCHUNK = 32

ROUTER_TILE = 512
N_RT = T // ROUTER_TILE


def _k1_router(x_ref, gate_w_ref, rout_ref, cnt_ref, cacc_ref):
    i = pl.program_id(0)
    x = x_ref[...]
    lt = lax.dot_general(gate_w_ref[...], x, (((0,), (1,)), ((), ())),
                         preferred_element_type=jnp.float32)
    m1 = jnp.max(lt, axis=0, keepdims=True)
    i1 = jnp.argmax(lt, axis=0, keepdims=True)
    eids = lax.broadcasted_iota(jnp.int32, lt.shape, 0)
    masked = jnp.where(eids == i1, -jnp.inf, lt)
    m2 = jnp.max(masked, axis=0, keepdims=True)
    i2 = jnp.argmax(masked, axis=0, keepdims=True)
    p1 = 1.0 / (1.0 + jnp.exp(m2 - m1))
    rout_ref[...] = jnp.concatenate(
        [i1.astype(jnp.float32), i2.astype(jnp.float32), p1, 1.0 - p1], axis=0)

    oh = (jnp.where(eids == i1, 1.0, 0.0)
          + jnp.where(eids == i2, 1.0, 0.0))
    ct = jnp.sum(oh, axis=1, keepdims=True)

    @pl.when(i == 0)
    def _():
        cacc_ref[...] = jnp.zeros_like(cacc_ref)

    cacc_ref[:, 0:1] += ct
    cnt_ref[...] = cacc_ref[:, 0:1]


def _k_shared(x_ref, sg_ref, su_ref, sd_ref, shared_ref):
    x = x_ref[...]
    sg = jnp.dot(x, sg_ref[...], preferred_element_type=jnp.float32)
    su = jnp.dot(x, su_ref[...], preferred_element_type=jnp.float32)
    hmid = (sg * lax.logistic(sg)) * su
    shared_ref[...] = jnp.dot(hmid, sd_ref[...],
                              preferred_element_type=jnp.float32)


def _scan_lanes(v, n):
    sh = 1
    while sh < n:
        z = jnp.zeros(v.shape[:-1] + (sh,), v.dtype)
        v = v + jnp.concatenate([z, v[..., :-sh]], axis=-1)
        sh *= 2
    return v


def _scan_sublanes(v, n):
    sh = 1
    while sh < n:
        z = jnp.zeros((sh,) + v.shape[1:], v.dtype)
        v = v + jnp.concatenate([z, v[:-sh]], axis=0)
        sh *= 2
    return v


def _k1b_dispatch(rout_ref, cnt_ref, pos_ref, te_ref, carry_ref):
    i = pl.program_id(0)
    c = cnt_ref[...]
    rounded = jnp.floor((c + (TILE_M - 1.0)) * (1.0 / TILE_M)) * TILE_M
    csum = _scan_sublanes(rounded, E)
    excl = csum - rounded

    i1 = rout_ref[0:1, :]
    i2 = rout_ref[1:2, :]
    eids = lax.broadcasted_iota(jnp.int32, (E, ROUTER_TILE), 0).astype(jnp.float32)
    oh0 = jnp.where(eids == i1, 1.0, 0.0)
    oh1 = jnp.where(eids == i2, 1.0, 0.0)

    @pl.when(i == 0)
    def _():
        carry_ref[...] = jnp.zeros_like(carry_ref)
        carry_ref[:, 0:1] += excl

    carry = carry_ref[:, 0:1]
    cs0 = _scan_lanes(oh0, ROUTER_TILE)
    cs1 = _scan_lanes(oh1, ROUTER_TILE)
    m = carry + (cs0 - oh0) + (cs1 - oh1)
    d0 = jnp.sum(oh0 * m, axis=0, keepdims=True)
    d1 = jnp.sum(oh1 * (m + oh0), axis=0, keepdims=True)
    pos_ref[...] = jnp.concatenate([d0, d1], axis=0).astype(jnp.int32)
    carry_ref[:, 0:1] += jnp.sum(oh0 + oh1, axis=1, keepdims=True)

    @pl.when(i == 0)
    def _():
        tcol = lax.broadcasted_iota(jnp.int32, (1, TE_LEN), 1).astype(jnp.float32) * TILE_M
        acc = jnp.zeros((1, TE_LEN))
        for e in range(E):
            acc = acc + jnp.where(tcol >= csum[e:e + 1, 0:1], 1.0, 0.0)
        te_ref[...] = jnp.minimum(acc, E - 1.0).astype(jnp.int32)


def _scb_rowscatter(x_hbm, rout_hbm, pos_hbm, xs_hbm, prob_hbm,
                    pos0_v, pos1_v, xrows_v, pb0_v, pb1_v, semA, semB):
    w = lax.axis_index("s") * 2 + lax.axis_index("c")
    pltpu.sync_copy(pos_hbm.at[0, pl.ds(w * TW3, TW3)], pos0_v)
    pltpu.sync_copy(pos_hbm.at[1, pl.ds(w * TW3, TW3)], pos1_v)
    pltpu.sync_copy(x_hbm.at[pl.ds(w * TW3, TW3)], xrows_v)
    a1 = pltpu.async_copy(xrows_v, xs_hbm.at[pos0_v], semA)
    a2 = pltpu.async_copy(xrows_v, xs_hbm.at[pos1_v], semB)
    a1.wait()
    a2.wait()
    pltpu.sync_copy(rout_hbm.at[2, pl.ds(w * TW3, TW3)], pb0_v)
    pltpu.sync_copy(rout_hbm.at[3, pl.ds(w * TW3, TW3)], pb1_v)
    a3 = pltpu.async_copy(pb0_v, prob_hbm.at[pos0_v], semA)
    a4 = pltpu.async_copy(pb1_v, prob_hbm.at[pos1_v], semB)
    a3.wait()
    a4.wait()


def _scc_combine(outs_hbm, pos_hbm, shared_hbm, y_hbm,
                 pos0_v, pos1_v, r0_v, r1_v, sh_v, semA, semB, semC):
    w = lax.axis_index("s") * 2 + lax.axis_index("c")
    pltpu.sync_copy(pos_hbm.at[0, pl.ds(w * TW3, TW3)], pos0_v)
    pltpu.sync_copy(pos_hbm.at[1, pl.ds(w * TW3, TW3)], pos1_v)
    for c in range(TW3 // CCHUNK):
        a1 = pltpu.async_copy(outs_hbm.at[pos0_v.at[pl.ds(c * CCHUNK, CCHUNK)]],
                              r0_v, semA)
        a2 = pltpu.async_copy(outs_hbm.at[pos1_v.at[pl.ds(c * CCHUNK, CCHUNK)]],
                              r1_v, semB)
        a3 = pltpu.async_copy(
            shared_hbm.at[pl.ds(w * TW3 + c * CCHUNK, CCHUNK)], sh_v, semC)
        a1.wait()
        a2.wait()
        a3.wait()

        def row_body(r, _):
            for k in range(H // 16):
                sl = pl.ds(k * 16, 16)
                r0_v[r, sl] = r0_v[r, sl] + r1_v[r, sl] + sh_v[r, sl]
            return 0

        lax.fori_loop(0, CCHUNK, row_body, 0)
        pltpu.sync_copy(r0_v, y_hbm.at[pl.ds(w * TW3 + c * CCHUNK, CCHUNK)])


def _k2_grouped(te_ref, xs_ref, wg_ref, wu_ref, wd_ref, prob_ref, out_ref):
    xs = xs_ref[...]
    g = jnp.dot(xs, wg_ref[0], preferred_element_type=jnp.float32)
    u = jnp.dot(xs, wu_ref[0], preferred_element_type=jnp.float32)
    hmid = (g * lax.logistic(g)) * u
    eo = jnp.dot(hmid, wd_ref[0], preferred_element_type=jnp.float32)
    out_ref[...] = eo * prob_ref[...]


@jax.jit
def kernel(hidden_states, gate_w, w_gate, w_up, w_down, s_gate, s_up, s_down):
    b, s, h = hidden_states.shape
    x = hidden_states.reshape(-1, h)

    rout, cnts = pl.pallas_call(
        _k1_router,
        grid=(N_RT,),
        in_specs=[
            pl.BlockSpec((ROUTER_TILE, H), lambda t: (t, 0)),
            pl.BlockSpec((H, E), lambda t: (0, 0)),
        ],
        out_specs=[
            pl.BlockSpec((4, ROUTER_TILE), lambda t: (0, t)),
            pl.BlockSpec((E, 1), lambda t: (0, 0)),
        ],
        out_shape=[
            jax.ShapeDtypeStruct((4, T), jnp.float32),
            jax.ShapeDtypeStruct((E, 1), jnp.float32),
        ],
        scratch_shapes=[pltpu.VMEM((E, 128), jnp.float32)],
    )(x, gate_w)

    shared = pl.pallas_call(
        _k_shared,
        grid=(N_RT,),
        in_specs=[
            pl.BlockSpec((ROUTER_TILE, H), lambda t: (t, 0)),
            pl.BlockSpec((H, S_INT), lambda t: (0, 0)),
            pl.BlockSpec((H, S_INT), lambda t: (0, 0)),
            pl.BlockSpec((S_INT, H), lambda t: (0, 0)),
        ],
        out_specs=pl.BlockSpec((ROUTER_TILE, H), lambda t: (t, 0)),
        out_shape=jax.ShapeDtypeStruct((T, H), jnp.float32),
    )(x, s_gate, s_up, s_down)

    pos, te = pl.pallas_call(
        _k1b_dispatch,
        grid=(N_RT,),
        in_specs=[
            pl.BlockSpec((4, ROUTER_TILE), lambda t: (0, t)),
            pl.BlockSpec((E, 1), lambda t: (0, 0)),
        ],
        out_specs=[
            pl.BlockSpec((2, ROUTER_TILE), lambda t: (0, t)),
            pl.BlockSpec((1, TE_LEN), lambda t: (0, 0)),
        ],
        out_shape=[
            jax.ShapeDtypeStruct((2, T), jnp.int32),
            jax.ShapeDtypeStruct((1, TE_LEN), jnp.int32),
        ],
        scratch_shapes=[pltpu.VMEM((E, 128), jnp.float32)],
    )(rout, cnts)

    mesh2 = plsc.VectorSubcoreMesh(core_axis_name="c", subcore_axis_name="s")
    rowscatter = pl.kernel(
        _scb_rowscatter,
        out_type=(
            jax.ShapeDtypeStruct((P, H), jnp.float32),
            jax.ShapeDtypeStruct((P,), jnp.float32),
        ),
        mesh=mesh2,
        compiler_params=pltpu.CompilerParams(needs_layout_passes=False),
        scratch_types=[
            pltpu.VMEM((TW3,), jnp.int32),
            pltpu.VMEM((TW3,), jnp.int32),
            pltpu.VMEM((TW3, H), jnp.float32),
            pltpu.VMEM((TW3,), jnp.float32),
            pltpu.VMEM((TW3,), jnp.float32),
            pltpu.SemaphoreType.DMA,
            pltpu.SemaphoreType.DMA,
        ],
    )
    xs, prob_sorted = rowscatter(x, rout, pos)

    outs = pl.pallas_call(
        _k2_grouped,
        grid_spec=pltpu.PrefetchScalarGridSpec(
            num_scalar_prefetch=1,
            grid=(NT2,),
            in_specs=[
                pl.BlockSpec((TILE_M, H), lambda i, te: (i, 0)),
                pl.BlockSpec((1, H, F_INT), lambda i, te: (te[i], 0, 0)),
                pl.BlockSpec((1, H, F_INT), lambda i, te: (te[i], 0, 0)),
                pl.BlockSpec((1, F_INT, H), lambda i, te: (te[i], 0, 0)),
                pl.BlockSpec((TILE_M, 1), lambda i, te: (i, 0)),
            ],
            out_specs=pl.BlockSpec((TILE_M, H), lambda i, te: (i, 0)),
        ),
        out_shape=jax.ShapeDtypeStruct((P, H), jnp.float32),
    )(te.reshape(TE_LEN), xs, w_gate, w_up, w_down, prob_sorted.reshape(P, 1))

    combine = pl.kernel(
        _scc_combine,
        out_type=jax.ShapeDtypeStruct((T, H), jnp.float32),
        mesh=mesh2,
        compiler_params=pltpu.CompilerParams(needs_layout_passes=False),
        scratch_types=[
            pltpu.VMEM((TW3,), jnp.int32),
            pltpu.VMEM((TW3,), jnp.int32),
            pltpu.VMEM((CCHUNK, H), jnp.float32),
            pltpu.VMEM((CCHUNK, H), jnp.float32),
            pltpu.VMEM((CCHUNK, H), jnp.float32),
            pltpu.SemaphoreType.DMA,
            pltpu.SemaphoreType.DMA,
            pltpu.SemaphoreType.DMA,
        ],
    )
    y = combine(outs, pos, shared)
    return y.reshape(b, s, h)


# speedup vs baseline: 1.6318x; 1.0137x over previous
import jax
import jax.numpy as jnp
from jax import lax
from jax.experimental import pallas as pl
from jax.experimental.pallas import tpu as pltpu
from jax.experimental.pallas import tpu_sc as plsc

B, S, H = 1, 2048, 768
E = 8
F_INT = 1024
S_INT = 512

T = B * S
TILE_M = 128
P = T * 2 + E * TILE_M
NPAD = P - 2 * T
NT2 = P // TILE_M
TE_LEN = 48

NW1 = 16
TW1 = T // NW1
NW2 = 32
RW2 = P // NW2
GCHUNK = 80
TW3 = T // NW2
CCHUNK = 32

ROUTER_TILE = 512
N_RT = T // ROUTER_TILE


def _k1_router(x_ref, gate_w_ref, rout_ref, cnt_ref, cacc_ref):
    i = pl.program_id(0)
    x = x_ref[...]
    lt = lax.dot_general(gate_w_ref[...], x, (((0,), (1,)), ((), ())),
                         preferred_element_type=jnp.float32)
    m1 = jnp.max(lt, axis=0, keepdims=True)
    i1 = jnp.argmax(lt, axis=0, keepdims=True)
    eids = lax.broadcasted_iota(jnp.int32, lt.shape, 0)
    masked = jnp.where(eids == i1, -jnp.inf, lt)
    m2 = jnp.max(masked, axis=0, keepdims=True)
    i2 = jnp.argmax(masked, axis=0, keepdims=True)
    p1 = 1.0 / (1.0 + jnp.exp(m2 - m1))
    rout_ref[...] = jnp.concatenate(
        [i1.astype(jnp.float32), i2.astype(jnp.float32), p1, 1.0 - p1], axis=0)

    oh = (jnp.where(eids == i1, 1.0, 0.0)
          + jnp.where(eids == i2, 1.0, 0.0))
    ct = jnp.sum(oh, axis=1, keepdims=True)

    @pl.when(i == 0)
    def _():
        cacc_ref[...] = jnp.zeros_like(cacc_ref)

    cacc_ref[:, 0:1] += ct
    cnt_ref[...] = cacc_ref[:, 0:1]


def _k_shared(x_ref, sg_ref, su_ref, sd_ref, shared_ref):
    x = x_ref[...]
    sg = jnp.dot(x, sg_ref[...], preferred_element_type=jnp.float32)
    su = jnp.dot(x, su_ref[...], preferred_element_type=jnp.float32)
    hmid = (sg * lax.logistic(sg)) * su
    shared_ref[...] = jnp.dot(hmid, sd_ref[...],
                              preferred_element_type=jnp.float32)


def _scan_lanes(v, n):
    sh = 1
    while sh < n:
        z = jnp.zeros(v.shape[:-1] + (sh,), v.dtype)
        v = v + jnp.concatenate([z, v[..., :-sh]], axis=-1)
        sh *= 2
    return v


def _scan_sublanes(v, n):
    sh = 1
    while sh < n:
        z = jnp.zeros((sh,) + v.shape[1:], v.dtype)
        v = v + jnp.concatenate([z, v[:-sh]], axis=0)
        sh *= 2
    return v


def _k1b_dispatch(rout_ref, cnt_ref, pos_ref, te_ref, carry_ref):
    i = pl.program_id(0)
    c = cnt_ref[...]
    rounded = jnp.floor((c + (TILE_M - 1.0)) * (1.0 / TILE_M)) * TILE_M
    csum = _scan_sublanes(rounded, E)
    excl = csum - rounded

    i1 = rout_ref[0:1, :]
    i2 = rout_ref[1:2, :]
    eids = lax.broadcasted_iota(jnp.int32, (E, ROUTER_TILE), 0).astype(jnp.float32)
    oh0 = jnp.where(eids == i1, 1.0, 0.0)
    oh1 = jnp.where(eids == i2, 1.0, 0.0)

    @pl.when(i == 0)
    def _():
        carry_ref[...] = jnp.zeros_like(carry_ref)
        carry_ref[:, 0:1] += excl

    carry = carry_ref[:, 0:1]
    cs0 = _scan_lanes(oh0, ROUTER_TILE)
    cs1 = _scan_lanes(oh1, ROUTER_TILE)
    m = carry + (cs0 - oh0) + (cs1 - oh1)
    d0 = jnp.sum(oh0 * m, axis=0, keepdims=True)
    d1 = jnp.sum(oh1 * (m + oh0), axis=0, keepdims=True)
    pos_ref[...] = jnp.concatenate([d0, d1], axis=0).astype(jnp.int32)
    carry_ref[:, 0:1] += jnp.sum(oh0 + oh1, axis=1, keepdims=True)

    @pl.when(i == 0)
    def _():
        tcol = lax.broadcasted_iota(jnp.int32, (1, TE_LEN), 1).astype(jnp.float32) * TILE_M
        acc = jnp.zeros((1, TE_LEN))
        for e in range(E):
            acc = acc + jnp.where(tcol >= csum[e:e + 1, 0:1], 1.0, 0.0)
        te_ref[...] = jnp.minimum(acc, E - 1.0).astype(jnp.int32)


def _scb_rowscatter(x_hbm, rout_hbm, pos_hbm, xs_hbm, prob_hbm,
                    posb_v, xrows_v, pb_v, semA, semB, semC):
    w = lax.axis_index("s") * 2 + lax.axis_index("c")
    c0 = pltpu.async_copy(pos_hbm.at[0, pl.ds(w * TW3, TW3)], posb_v.at[0],
                          semA)
    c1 = pltpu.async_copy(pos_hbm.at[1, pl.ds(w * TW3, TW3)], posb_v.at[1],
                          semB)
    c2 = pltpu.async_copy(x_hbm.at[pl.ds(w * TW3, TW3)], xrows_v, semC)
    pltpu.sync_copy(rout_hbm.at[2, pl.ds(w * TW3, TW3)], pb_v.at[0])
    pltpu.sync_copy(rout_hbm.at[3, pl.ds(w * TW3, TW3)], pb_v.at[1])
    c0.wait()
    c1.wait()
    c2.wait()
    a1 = pltpu.async_copy(xrows_v, xs_hbm.at[posb_v.at[0]], semA)
    a2 = pltpu.async_copy(xrows_v, xs_hbm.at[posb_v.at[1]], semB)
    a3 = pltpu.async_copy(pb_v.at[0], prob_hbm.at[posb_v.at[0]], semC)
    a1.wait()
    a2.wait()
    a3.wait()
    a4 = pltpu.async_copy(pb_v.at[1], prob_hbm.at[posb_v.at[1]], semA)
    a4.wait()


def _scc_combine(outs_hbm, pos_hbm, shared_hbm, y_hbm,
                 pos0_v, pos1_v, r0_v, r1_v, sh_v, semA, semB, semC):
    w = lax.axis_index("s") * 2 + lax.axis_index("c")
    pltpu.sync_copy(pos_hbm.at[0, pl.ds(w * TW3, TW3)], pos0_v)
    pltpu.sync_copy(pos_hbm.at[1, pl.ds(w * TW3, TW3)], pos1_v)
    for c in range(TW3 // CCHUNK):
        a1 = pltpu.async_copy(outs_hbm.at[pos0_v.at[pl.ds(c * CCHUNK, CCHUNK)]],
                              r0_v, semA)
        a2 = pltpu.async_copy(outs_hbm.at[pos1_v.at[pl.ds(c * CCHUNK, CCHUNK)]],
                              r1_v, semB)
        a3 = pltpu.async_copy(
            shared_hbm.at[pl.ds(w * TW3 + c * CCHUNK, CCHUNK)], sh_v, semC)
        a1.wait()
        a2.wait()
        a3.wait()

        def row_body(r, _):
            for k in range(H // 16):
                sl = pl.ds(k * 16, 16)
                r0_v[r, sl] = r0_v[r, sl] + r1_v[r, sl] + sh_v[r, sl]
            return 0

        lax.fori_loop(0, CCHUNK, row_body, 0)
        pltpu.sync_copy(r0_v, y_hbm.at[pl.ds(w * TW3 + c * CCHUNK, CCHUNK)])


def _k2_grouped(te_ref, xs_ref, wg_ref, wu_ref, wd_ref, prob_ref, out_ref):
    xs = xs_ref[...]
    g = jnp.dot(xs, wg_ref[0], preferred_element_type=jnp.float32)
    u = jnp.dot(xs, wu_ref[0], preferred_element_type=jnp.float32)
    hmid = (g * lax.logistic(g)) * u
    eo = jnp.dot(hmid, wd_ref[0], preferred_element_type=jnp.float32)
    out_ref[...] = eo * prob_ref[...]


@jax.jit
def kernel(hidden_states, gate_w, w_gate, w_up, w_down, s_gate, s_up, s_down):
    b, s, h = hidden_states.shape
    x = hidden_states.reshape(-1, h)

    rout, cnts = pl.pallas_call(
        _k1_router,
        grid=(N_RT,),
        in_specs=[
            pl.BlockSpec((ROUTER_TILE, H), lambda t: (t, 0)),
            pl.BlockSpec((H, E), lambda t: (0, 0)),
        ],
        out_specs=[
            pl.BlockSpec((4, ROUTER_TILE), lambda t: (0, t)),
            pl.BlockSpec((E, 1), lambda t: (0, 0)),
        ],
        out_shape=[
            jax.ShapeDtypeStruct((4, T), jnp.float32),
            jax.ShapeDtypeStruct((E, 1), jnp.float32),
        ],
        scratch_shapes=[pltpu.VMEM((E, 128), jnp.float32)],
    )(x, gate_w)

    shared = pl.pallas_call(
        _k_shared,
        grid=(N_RT,),
        in_specs=[
            pl.BlockSpec((ROUTER_TILE, H), lambda t: (t, 0)),
            pl.BlockSpec((H, S_INT), lambda t: (0, 0)),
            pl.BlockSpec((H, S_INT), lambda t: (0, 0)),
            pl.BlockSpec((S_INT, H), lambda t: (0, 0)),
        ],
        out_specs=pl.BlockSpec((ROUTER_TILE, H), lambda t: (t, 0)),
        out_shape=jax.ShapeDtypeStruct((T, H), jnp.float32),
    )(x, s_gate, s_up, s_down)

    pos, te = pl.pallas_call(
        _k1b_dispatch,
        grid=(N_RT,),
        in_specs=[
            pl.BlockSpec((4, ROUTER_TILE), lambda t: (0, t)),
            pl.BlockSpec((E, 1), lambda t: (0, 0)),
        ],
        out_specs=[
            pl.BlockSpec((2, ROUTER_TILE), lambda t: (0, t)),
            pl.BlockSpec((1, TE_LEN), lambda t: (0, 0)),
        ],
        out_shape=[
            jax.ShapeDtypeStruct((2, T), jnp.int32),
            jax.ShapeDtypeStruct((1, TE_LEN), jnp.int32),
        ],
        scratch_shapes=[pltpu.VMEM((E, 128), jnp.float32)],
    )(rout, cnts)

    mesh2 = plsc.VectorSubcoreMesh(core_axis_name="c", subcore_axis_name="s")
    rowscatter = pl.kernel(
        _scb_rowscatter,
        out_type=(
            jax.ShapeDtypeStruct((P, H), jnp.float32),
            jax.ShapeDtypeStruct((P,), jnp.float32),
        ),
        mesh=mesh2,
        compiler_params=pltpu.CompilerParams(needs_layout_passes=False),
        scratch_types=[
            pltpu.VMEM((2, TW3), jnp.int32),
            pltpu.VMEM((TW3, H), jnp.float32),
            pltpu.VMEM((2, TW3), jnp.float32),
            pltpu.SemaphoreType.DMA,
            pltpu.SemaphoreType.DMA,
            pltpu.SemaphoreType.DMA,
        ],
    )
    xs, prob_sorted = rowscatter(x, rout, pos)

    outs = pl.pallas_call(
        _k2_grouped,
        grid_spec=pltpu.PrefetchScalarGridSpec(
            num_scalar_prefetch=1,
            grid=(NT2,),
            in_specs=[
                pl.BlockSpec((TILE_M, H), lambda i, te: (i, 0)),
                pl.BlockSpec((1, H, F_INT), lambda i, te: (te[i], 0, 0)),
                pl.BlockSpec((1, H, F_INT), lambda i, te: (te[i], 0, 0)),
                pl.BlockSpec((1, F_INT, H), lambda i, te: (te[i], 0, 0)),
                pl.BlockSpec((TILE_M, 1), lambda i, te: (i, 0)),
            ],
            out_specs=pl.BlockSpec((TILE_M, H), lambda i, te: (i, 0)),
        ),
        out_shape=jax.ShapeDtypeStruct((P, H), jnp.float32),
    )(te.reshape(TE_LEN), xs, w_gate, w_up, w_down, prob_sorted.reshape(P, 1))

    combine = pl.kernel(
        _scc_combine,
        out_type=jax.ShapeDtypeStruct((T, H), jnp.float32),
        mesh=mesh2,
        compiler_params=pltpu.CompilerParams(needs_layout_passes=False),
        scratch_types=[
            pltpu.VMEM((TW3,), jnp.int32),
            pltpu.VMEM((TW3,), jnp.int32),
            pltpu.VMEM((CCHUNK, H), jnp.float32),
            pltpu.VMEM((CCHUNK, H), jnp.float32),
            pltpu.VMEM((CCHUNK, H), jnp.float32),
            pltpu.SemaphoreType.DMA,
            pltpu.SemaphoreType.DMA,
            pltpu.SemaphoreType.DMA,
        ],
    )
    y = combine(outs, pos, shared)
    return y.reshape(b, s, h)
